# baseline (speedup 1.0000x reference)
import functools

import jax
import jax.numpy as jnp
from jax import lax
from jax.experimental import pallas as pl
from jax.experimental.pallas import tpu as pltpu
from jax.experimental.pallas import tpu_sc as plsc

N_ROWS = 1_000_000
EMB_DIM = 32
BATCH = 16384

_info = plsc.get_sparse_core_info()
_NC, _NS = _info.num_cores, _info.num_subcores
_NW = _NC * _NS
_BPW = BATCH // _NW
_UNROLL = 8


def _build():
    mesh = plsc.VectorSubcoreMesh(core_axis_name="c", subcore_axis_name="s")
    out_sds = jax.ShapeDtypeStruct((BATCH, EMB_DIM), jnp.float32)

    @functools.partial(
        pl.kernel,
        out_type=(out_sds, out_sds, out_sds),
        mesh=mesh,
        scratch_types=[
            pltpu.SMEM((3 * _BPW,), jnp.int32),
            pltpu.VMEM_SHARED((_NS * 3 * _BPW,), jnp.int32),
            pltpu.SemaphoreType.DMA,
        ],
    )
    def gather3(table, u_hbm, p_hbm, n_hbm, out_u, out_p, out_n, idx_s,
                idx_sh, sem):
        cid = lax.axis_index("c")
        sid = lax.axis_index("s")
        wid = sid * _NC + cid
        base = wid * _BPW
        in_refs = (u_hbm, p_hbm, n_hbm)
        out_refs = (out_u, out_p, out_n)

        sh_base = sid * (3 * _BPW)
        for j in range(3):
            pltpu.sync_copy(
                in_refs[j].at[pl.ds(base, _BPW)],
                idx_sh.at[pl.ds(sh_base + j * _BPW, _BPW)],
            )
        pltpu.sync_copy(idx_sh.at[pl.ds(sh_base, 3 * _BPW)], idx_s)

        for j in range(3):
            out_ref = out_refs[j]

            def body(i, _, j=j, out_ref=out_ref):
                for k in range(_UNROLL):
                    o = i * _UNROLL + k
                    r = idx_s[j * _BPW + o]
                    pltpu.make_async_copy(
                        table.at[pl.ds(r, 1)],
                        out_ref.at[pl.ds(base + o, 1)],
                        sem,
                    ).start()
                return 0

            lax.fori_loop(0, _BPW // _UNROLL, body, 0)

        for j in range(3):
            pltpu.make_async_copy(
                table.at[pl.ds(0, _BPW)],
                out_refs[j].at[pl.ds(base, _BPW)],
                sem,
            ).wait()

    return gather3


_gather3 = _build()


def kernel(embeds, users, pos_items, neg_items):
    u, p, n = _gather3(embeds, users, pos_items, neg_items)
    return (u, p, n, u, p, n)


# speedup vs baseline: 3.1604x; 3.1604x over previous
import functools

import jax
import jax.numpy as jnp
from jax import lax
from jax.experimental import pallas as pl
from jax.experimental.pallas import tpu as pltpu
from jax.experimental.pallas import tpu_sc as plsc

N_ROWS = 1_000_000
EMB_DIM = 32
BATCH = 16384

_info = plsc.get_sparse_core_info()
_NC, _NS = _info.num_cores, _info.num_subcores
_NW = _NC * _NS
_BPW = BATCH // _NW
_CHUNK = 256
_NCHUNKS = 3 * _BPW // _CHUNK
_NBUF = 3
_UNROLL = 8


def _build():
    mesh = plsc.VectorSubcoreMesh(core_axis_name="c", subcore_axis_name="s")
    out_sds = jax.ShapeDtypeStruct((BATCH, EMB_DIM), jnp.float32)

    @functools.partial(
        pl.kernel,
        out_type=(out_sds, out_sds, out_sds),
        mesh=mesh,
        scratch_types=[
            pltpu.SMEM((3 * _BPW,), jnp.int32),
            pltpu.VMEM_SHARED((_NS * 3 * _BPW,), jnp.int32),
            [pltpu.VMEM((_CHUNK, EMB_DIM), jnp.float32) for _ in range(_NBUF)],
            [pltpu.SemaphoreType.DMA for _ in range(_NBUF)],
            [pltpu.SemaphoreType.DMA for _ in range(_NBUF)],
        ],
    )
    def gather3(table, u_hbm, p_hbm, n_hbm, out_u, out_p, out_n, idx_s,
                idx_sh, bufs, sem_g, sem_s):
        cid = lax.axis_index("c")
        sid = lax.axis_index("s")
        wid = sid * _NC + cid
        base = wid * _BPW
        in_refs = (u_hbm, p_hbm, n_hbm)
        out_refs = (out_u, out_p, out_n)

        sh_base = sid * (3 * _BPW)
        for j in range(3):
            pltpu.sync_copy(
                in_refs[j].at[pl.ds(base, _BPW)],
                idx_sh.at[pl.ds(sh_base + j * _BPW, _BPW)],
            )
        pltpu.sync_copy(idx_sh.at[pl.ds(sh_base, 3 * _BPW)], idx_s)

        def fire(c):
            b = c % _NBUF
            buf = bufs[b]

            def body(i, _):
                for k in range(_UNROLL):
                    o = i * _UNROLL + k
                    r = idx_s[c * _CHUNK + o]
                    pltpu.make_async_copy(
                        table.at[pl.ds(r, 1)],
                        buf.at[pl.ds(o, 1)],
                        sem_g[b],
                    ).start()
                return 0

            lax.fori_loop(0, _CHUNK // _UNROLL, body, 0)

        def drain_and_store(c):
            b = c % _NBUF
            j, h = divmod(c, _BPW // _CHUNK)
            pltpu.make_async_copy(
                table.at[pl.ds(0, _CHUNK)], bufs[b], sem_g[b]
            ).wait()
            return pltpu.async_copy(
                bufs[b],
                out_refs[j].at[pl.ds(base + h * _CHUNK, _CHUNK)],
                sem_s[b],
            )

        stores = [None] * _NCHUNKS
        fire(0)
        for c in range(1, _NCHUNKS):
            if c >= _NBUF:
                stores[c - _NBUF].wait()
            fire(c)
            stores[c - 1] = drain_and_store(c - 1)
        stores[_NCHUNKS - 1] = drain_and_store(_NCHUNKS - 1)
        for c in range(_NCHUNKS - _NBUF, _NCHUNKS):
            stores[c].wait()

    return gather3


_gather3 = _build()


def kernel(embeds, users, pos_items, neg_items):
    u, p, n = _gather3(embeds, users, pos_items, neg_items)
    return (u, p, n, u, p, n)
